# Initial kernel scaffold; baseline (speedup 1.0000x reference)
import jax
import jax.numpy as jnp
from jax.experimental import pallas as pl


def kernel(embed, adj_sp_norm, edge_index, edge_weight, deg):
    raise NotImplementedError("write your pallas kernel here")


# speedup vs baseline: 2.6495x; 2.6495x over previous
import functools

import jax
import jax.numpy as jnp
from jax import lax
from jax.experimental import pallas as pl
from jax.experimental.pallas import tpu as pltpu
from jax.experimental.pallas import tpu_sc as plsc

N_NODES = 10000
D = 128
E = 320000
NC = 2
NS = 16
NW = NC * NS
CHUNK = 128
NCHUNK = -(-E // (NW * CHUNK))
EPW = NCHUNK * CHUNK
E_PAD = NW * EPW
N_PAD = 10240
ROWS_PER_TILE = N_PAD // NS
ZROWS = 128


def _hop_body(src_hbm, row_hbm, col_hbm, w_hbm, parts_hbm,
              acc_sh, rows_v, zbuf_v, ridx_v, cidx_v, w_v, gsem):
  cid = lax.axis_index("c")
  sid = lax.axis_index("s")
  wid = sid * NC + cid

  def _zero_row(r, _):
    for j in range(D // 16):
      zbuf_v[r, pl.ds(j * 16, 16)] = jnp.zeros((16,), jnp.float32)
    return 0
  lax.fori_loop(0, ZROWS, _zero_row, 0)
  for k in range(ROWS_PER_TILE // ZROWS):
    pltpu.sync_copy(zbuf_v, acc_sh.at[pl.ds(sid * ROWS_PER_TILE + k * ZROWS,
                                            ZROWS)])
  plsc.subcore_barrier()

  def _chunk(i, _):
    base = wid * EPW + i * CHUNK
    pltpu.sync_copy(row_hbm.at[pl.ds(base, CHUNK)], ridx_v)
    pltpu.sync_copy(col_hbm.at[pl.ds(base, CHUNK)], cidx_v.at[0])
    pltpu.sync_copy(w_hbm.at[pl.ds(base, CHUNK)], w_v)
    pltpu.async_copy(src_hbm.at[ridx_v], rows_v, gsem).wait()

    def _scale(e, _):
      ws = plsc.load_gather(w_v, [jnp.full((16,), e, jnp.int32)])
      for j in range(D // 16):
        sl = pl.ds(j * 16, 16)
        rows_v[e, sl] = rows_v[e, sl] * ws
      return 0
    lax.fori_loop(0, CHUNK, _scale, 0)

    pltpu.sync_copy(rows_v, acc_sh.at[cidx_v.at[0]], add=True)
    return 0
  lax.fori_loop(0, NCHUNK, _chunk, 0)

  plsc.subcore_barrier()

  for k in range(ROWS_PER_TILE // ZROWS):
    off = sid * ROWS_PER_TILE + k * ZROWS
    pltpu.sync_copy(acc_sh.at[pl.ds(off, ZROWS)], zbuf_v)
    pltpu.sync_copy(zbuf_v, parts_hbm.at[cid, pl.ds(off, ZROWS)])


_hop = pl.kernel(
    _hop_body,
    out_type=jax.ShapeDtypeStruct((NC, N_PAD, D), jnp.float32),
    mesh=plsc.VectorSubcoreMesh(core_axis_name="c", subcore_axis_name="s",
                                num_cores=NC, num_subcores=NS),
    scratch_types=[
        pltpu.VMEM_SHARED((N_PAD, D), jnp.float32),
        pltpu.VMEM((CHUNK, D), jnp.float32),
        pltpu.VMEM((ZROWS, D), jnp.float32),
        pltpu.VMEM((CHUNK,), jnp.int32),
        pltpu.VMEM((1, CHUNK), jnp.int32),
        pltpu.VMEM((CHUNK,), jnp.float32),
        pltpu.SemaphoreType.DMA,
    ],
    compiler_params=pltpu.CompilerParams(needs_layout_passes=False),
)


def _add_body(a_ref, b_ref, o_ref):
  o_ref[...] = a_ref[...] + b_ref[...]


_BLK = 2000


def _combine(parts):
  return pl.pallas_call(
      _add_body,
      grid=(N_NODES // _BLK,),
      in_specs=[pl.BlockSpec((_BLK, D), lambda i: (i, 0)),
                pl.BlockSpec((_BLK, D), lambda i: (i, 0))],
      out_specs=pl.BlockSpec((_BLK, D), lambda i: (i, 0)),
      out_shape=jax.ShapeDtypeStruct((N_NODES, D), jnp.float32),
  )(parts[0, :N_NODES], parts[1, :N_NODES])


@jax.jit
def kernel(embed, adj_sp_norm, edge_index, edge_weight, deg):
  pad = E_PAD - E
  row = jnp.concatenate([edge_index[0], jnp.zeros((pad,), jnp.int32)])
  col = jnp.concatenate([edge_index[1], jnp.zeros((pad,), jnp.int32)])
  w = jnp.concatenate([edge_weight, jnp.zeros((pad,), jnp.float32)])

  embs = [embed]
  a = embed
  for _ in range(3):
    parts = _hop(a, row, col, w)
    a = _combine(parts)
    embs.append(a)
  embs = jnp.stack(embs, axis=1)
  return (embs[: N_NODES // 2], embs[N_NODES // 2:])


# speedup vs baseline: 3.0292x; 1.1433x over previous
import functools

import jax
import jax.numpy as jnp
from jax import lax
from jax.experimental import pallas as pl
from jax.experimental.pallas import tpu as pltpu
from jax.experimental.pallas import tpu_sc as plsc

N_NODES = 10000
D = 128
E = 320000
NC = 2
NS = 16
NW = NC * NS
CHUNK = 128
NCHUNK = 80
EPW = NCHUNK * CHUNK
E_PAD = NW * EPW
N_PAD = 10240
ROWS_PER_TILE = N_PAD // NS
ZROWS = 128


SDEPTH = 40
NSTAGE = NCHUNK // SDEPTH


def _hop_body(src_hbm, row_hbm, col_hbm, w_hbm, parts_hbm,
              acc_sh, rows0, rows1, ridx_v, cidx_v, w_v,
              gsem0, gsem1):
  cid = lax.axis_index("c")
  sid = lax.axis_index("s")
  wid = sid * NC + cid

  def _zero_row(r, _):
    for j in range(D // 16):
      rows0[r, pl.ds(j * 16, 16)] = jnp.zeros((16,), jnp.float32)
    return 0
  lax.fori_loop(0, ZROWS, _zero_row, 0)
  for k in range(ROWS_PER_TILE // ZROWS):
    pltpu.sync_copy(rows0, acc_sh.at[pl.ds(sid * ROWS_PER_TILE + k * ZROWS,
                                           ZROWS)])
  plsc.subcore_barrier()

  bufs = (rows0, rows1)
  sems = (gsem0, gsem1)

  for s in range(NSTAGE):
    cbase = wid * NCHUNK + s * SDEPTH
    pltpu.sync_copy(row_hbm.at[pl.ds(cbase, SDEPTH)], ridx_v)
    pltpu.sync_copy(col_hbm.at[pl.ds(cbase, SDEPTH)], cidx_v)
    pltpu.sync_copy(w_hbm.at[pl.ds(cbase, SDEPTH)], w_v)

    pltpu.async_copy(src_hbm.at[ridx_v.at[0]], rows0, gsem0)
    pltpu.async_copy(src_hbm.at[ridx_v.at[1]], rows1, gsem1)

    def _pair(k, _):
      for b in range(2):
        li = 2 * k + b
        buf, sem = bufs[b], sems[b]
        pltpu.make_async_copy(src_hbm.at[ridx_v.at[li]], buf, sem).wait()

        def _scale(e, _):
          ws = plsc.load_gather(w_v, [jnp.full((16,), li, jnp.int32),
                                      jnp.full((16,), e, jnp.int32)])
          for j in range(D // 16):
            sl = pl.ds(j * 16, 16)
            buf[e, sl] = buf[e, sl] * ws
          return 0
        lax.fori_loop(0, CHUNK, _scale, 0)

        pltpu.sync_copy(buf, acc_sh.at[cidx_v.at[li]], add=True)

        @pl.when(li + 2 < SDEPTH)
        def _():
          pltpu.async_copy(src_hbm.at[ridx_v.at[li + 2]], buf, sem)
      return 0
    lax.fori_loop(0, SDEPTH // 2, _pair, 0)

  plsc.subcore_barrier()

  for k in range(ROWS_PER_TILE // ZROWS):
    off = sid * ROWS_PER_TILE + k * ZROWS
    pltpu.sync_copy(acc_sh.at[pl.ds(off, ZROWS)], rows0)
    pltpu.sync_copy(rows0, parts_hbm.at[cid, pl.ds(off, ZROWS)])


_hop = pl.kernel(
    _hop_body,
    out_type=jax.ShapeDtypeStruct((NC, N_PAD, D), jnp.float32),
    mesh=plsc.VectorSubcoreMesh(core_axis_name="c", subcore_axis_name="s",
                                num_cores=NC, num_subcores=NS),
    scratch_types=[
        pltpu.VMEM_SHARED((N_PAD, D), jnp.float32),
        pltpu.VMEM((CHUNK, D), jnp.float32),
        pltpu.VMEM((CHUNK, D), jnp.float32),
        pltpu.VMEM((SDEPTH, CHUNK), jnp.int32),
        pltpu.VMEM((SDEPTH, CHUNK), jnp.int32),
        pltpu.VMEM((SDEPTH, CHUNK), jnp.float32),
        pltpu.SemaphoreType.DMA,
        pltpu.SemaphoreType.DMA,
    ],
    compiler_params=pltpu.CompilerParams(needs_layout_passes=False),
)


def _add_body(a_ref, b_ref, o_ref):
  o_ref[...] = a_ref[...] + b_ref[...]


_BLK = 2000


def _combine(parts):
  return pl.pallas_call(
      _add_body,
      grid=(N_NODES // _BLK,),
      in_specs=[pl.BlockSpec((_BLK, D), lambda i: (i, 0)),
                pl.BlockSpec((_BLK, D), lambda i: (i, 0))],
      out_specs=pl.BlockSpec((_BLK, D), lambda i: (i, 0)),
      out_shape=jax.ShapeDtypeStruct((N_NODES, D), jnp.float32),
  )(parts[0, :N_NODES], parts[1, :N_NODES])


@jax.jit
def kernel(embed, adj_sp_norm, edge_index, edge_weight, deg):
  pad = E_PAD - E
  row = jnp.concatenate([edge_index[0], jnp.zeros((pad,), jnp.int32)])
  col = jnp.concatenate([edge_index[1], jnp.zeros((pad,), jnp.int32)])
  w = jnp.concatenate([edge_weight, jnp.zeros((pad,), jnp.float32)])
  row2d = row.reshape(NW * NCHUNK, CHUNK)
  col2d = col.reshape(NW * NCHUNK, CHUNK)
  w2d = w.reshape(NW * NCHUNK, CHUNK)

  embs = [embed]
  a = embed
  for _ in range(3):
    parts = _hop(a, row2d, col2d, w2d)
    a = _combine(parts)
    embs.append(a)
  embs = jnp.stack(embs, axis=1)
  return (embs[: N_NODES // 2], embs[N_NODES // 2:])
